# baseline (speedup 1.0000x reference)
import functools

import jax
import jax.numpy as jnp
from jax import lax
from jax.experimental import pallas as pl
from jax.experimental.pallas import tpu as pltpu
from jax.experimental.pallas import tpu_sc as plsc

N = 10000
E = 160000
D = 256
H = 128
G = 64
NPAD = 10240
NS = 16
EC = 100
NCH = E // NS // EC
NCH2 = E // 2 // NS // EC
DW = 32
BN = 2000
GRID = N // BN

@functools.cache
def _sc_kernels():
    mesh = plsc.VectorSubcoreMesh(core_axis_name="c", subcore_axis_name="s")
    deg_fn = functools.partial(
        pl.kernel,
        out_type=jax.ShapeDtypeStruct((2 * NPAD, DW), jnp.float32),
        mesh=mesh,
        scratch_types=[
            pltpu.VMEM_SHARED((NPAD, DW), jnp.float32),
            pltpu.VMEM((NCH2, EC), jnp.int32),
            pltpu.VMEM((EC, DW), jnp.float32),
        ],
    )(_deg_body)
    spmm_fn = functools.partial(
        pl.kernel,
        out_type=jax.ShapeDtypeStruct((2 * NPAD, H), jnp.float32),
        mesh=mesh,
        scratch_types=[
            pltpu.VMEM_SHARED((NPAD, H), jnp.float32),
            pltpu.VMEM((NCH // 2, EC), jnp.int32),
            pltpu.VMEM((NCH // 2, EC), jnp.int32),
            pltpu.VMEM((EC, H), jnp.float32),
            pltpu.VMEM((EC, H), jnp.float32),
            pltpu.SemaphoreType.DMA,
            pltpu.SemaphoreType.DMA,
            pltpu.SemaphoreType.DMA,
            pltpu.SemaphoreType.DMA,
        ],
    )(_spmm_body)
    return deg_fn, spmm_fn


def _deg_body(dst_hbm, ones_hbm, zeros_hbm, deg_hbm, degw, idxd, ones_v):
    c = lax.axis_index("c")
    s = lax.axis_index("s")
    r0 = s * (NPAD // NS)
    pltpu.sync_copy(zeros_hbm, degw.at[pl.ds(r0, NPAD // NS)])
    pltpu.sync_copy(ones_hbm, ones_v)
    pltpu.sync_copy(dst_hbm.at[c, s], idxd)
    plsc.subcore_barrier()

    def chunk(j, carry):
        pltpu.sync_copy(ones_v, degw.at[idxd.at[j]], add=True)
        return carry

    lax.fori_loop(0, NCH2, chunk, 0)
    plsc.subcore_barrier()
    pltpu.sync_copy(degw.at[pl.ds(r0, NPAD // NS)],
                    deg_hbm.at[pl.ds(c * NPAD + r0, NPAD // NS)])


def _spmm_body(hs_hbm, srcoff_hbm, dst_hbm, out_hbm, agg, idxs, idxd, rows0,
               rows1, sem0, sem1, ssem0, ssem1):
    c = lax.axis_index("c")
    s = lax.axis_index("s")
    r0 = s * (NPAD // NS)
    base = c * NPAD
    def seed(i, carry):
        pltpu.sync_copy(hs_hbm.at[pl.ds(base + r0 + i * 128, 128)],
                        agg.at[pl.ds(r0 + i * 128, 128)])
        return carry

    lax.fori_loop(0, (NPAD // NS) // 128, seed, 0)
    plsc.subcore_barrier()

    HCH = NCH // 2
    for h in range(2):
        pltpu.sync_copy(srcoff_hbm.at[c, s, h], idxs)
        pltpu.sync_copy(dst_hbm.at[s, h], idxd)
        pltpu.async_copy(hs_hbm.at[idxs.at[0]], rows0, sem0)
        pltpu.async_copy(hs_hbm.at[idxs.at[1]], rows1, sem1)

        def pair(k, carry):
            j0 = 2 * k
            j1 = j0 + 1
            j2 = jnp.minimum(j0 + 2, HCH - 1)
            j3 = jnp.minimum(j0 + 3, HCH - 1)
            pltpu.make_async_copy(hs_hbm.at[idxs.at[j0]], rows0, sem0).wait()
            pltpu.make_async_copy(rows0, agg.at[idxd.at[j0]],
                                  ssem0).start(add=True)
            pltpu.make_async_copy(hs_hbm.at[idxs.at[j1]], rows1, sem1).wait()
            pltpu.make_async_copy(rows0, agg.at[idxd.at[j0]], ssem0).wait()
            pltpu.async_copy(hs_hbm.at[idxs.at[j2]], rows0, sem0)
            pltpu.make_async_copy(rows1, agg.at[idxd.at[j1]],
                                  ssem1).start(add=True)
            pltpu.make_async_copy(rows1, agg.at[idxd.at[j1]], ssem1).wait()
            pltpu.async_copy(hs_hbm.at[idxs.at[j3]], rows1, sem1)
            return carry

        lax.fori_loop(0, HCH // 2, pair, 0)
        pltpu.make_async_copy(hs_hbm.at[idxs.at[HCH - 1]], rows0, sem0).wait()
        pltpu.make_async_copy(hs_hbm.at[idxs.at[HCH - 1]], rows1, sem1).wait()
    plsc.subcore_barrier()

    def wb(i, carry):
        pltpu.sync_copy(agg.at[pl.ds(r0 + i * 128, 128)],
                        out_hbm.at[pl.ds(base + r0 + i * 128, 128)])
        return carry

    lax.fori_loop(0, (NPAD // NS) // 128, wb, 0)


def _embed_body(x_ref, we_ref, be_ref, w1_ref, out_ref):
    h0 = jnp.dot(x_ref[...], we_ref[...],
                 preferred_element_type=jnp.float32) + be_ref[...]
    t1 = jnp.dot(h0, w1_ref[...], preferred_element_type=jnp.float32)
    out_ref[0] = t1[:, :H]
    out_ref[1] = t1[:, H:]


_embed_fn = pl.pallas_call(
    _embed_body,
    grid=(GRID,),
    in_specs=[
        pl.BlockSpec((BN, D), lambda i: (i, 0)),
        pl.BlockSpec((D, D), lambda i: (0, 0)),
        pl.BlockSpec((1, D), lambda i: (0, 0)),
        pl.BlockSpec((D, D), lambda i: (0, 0)),
    ],
    out_specs=pl.BlockSpec((2, BN, H), lambda i: (0, i, 0)),
    out_shape=jax.ShapeDtypeStruct((2, NPAD, H), jnp.float32),
)


def _scale_body(t_ref, deg_ref, out_ref):
    dinv = lax.rsqrt(deg_ref[...] + 1.0)
    out_ref[0] = t_ref[0] * dinv
    out_ref[1] = t_ref[1] * dinv


_scale_fn = pl.pallas_call(
    _scale_body,
    grid=(GRID,),
    in_specs=[
        pl.BlockSpec((2, BN, H), lambda i: (0, i, 0)),
        pl.BlockSpec((BN, 1), lambda i: (i, 0)),
    ],
    out_specs=pl.BlockSpec((2, BN, H), lambda i: (0, i, 0)),
    out_shape=jax.ShapeDtypeStruct((2, NPAD, H), jnp.float32),
)


def _mid_body(agg_ref, deg_ref, b1_ref, w2_ref, out_ref):
    dinv = lax.rsqrt(deg_ref[...] + 1.0)
    a = agg_ref[...]
    h1 = jnp.concatenate([a[0], a[1]], axis=1)
    h1 = jax.nn.relu(h1 * dinv + b1_ref[...])
    t2 = jnp.dot(h1, w2_ref[...], preferred_element_type=jnp.float32)
    hs = t2 * dinv
    out_ref[0] = hs[:, :H]
    out_ref[1] = hs[:, H:]


_mid_fn = pl.pallas_call(
    _mid_body,
    grid=(GRID,),
    in_specs=[
        pl.BlockSpec((2, BN, H), lambda i: (0, i, 0)),
        pl.BlockSpec((BN, 1), lambda i: (i, 0)),
        pl.BlockSpec((1, D), lambda i: (0, 0)),
        pl.BlockSpec((D, D), lambda i: (0, 0)),
    ],
    out_specs=pl.BlockSpec((2, BN, H), lambda i: (0, i, 0)),
    out_shape=jax.ShapeDtypeStruct((2, NPAD, H), jnp.float32),
)


def _pool_body(agg_ref, deg_ref, b2_ref, batch_ref, out_ref):
    dinv = lax.rsqrt(deg_ref[...] + 1.0)
    a = agg_ref[...]
    h2 = jnp.concatenate([a[0], a[1]], axis=1)
    h2 = jax.nn.relu(h2 * dinv + b2_ref[...])
    gid = lax.broadcasted_iota(jnp.int32, (BN, G), 1)
    oh = (batch_ref[...] == gid).astype(jnp.float32)
    pooled = lax.dot_general(oh, h2, (((0,), (0,)), ((), ())),
                             preferred_element_type=jnp.float32)
    i = pl.program_id(0)

    @pl.when(i == 0)
    def _():
        out_ref[...] = pooled

    @pl.when(i > 0)
    def _():
        out_ref[...] += pooled


_pool_fn = pl.pallas_call(
    _pool_body,
    grid=(GRID,),
    in_specs=[
        pl.BlockSpec((2, BN, H), lambda i: (0, i, 0)),
        pl.BlockSpec((BN, 1), lambda i: (i, 0)),
        pl.BlockSpec((1, D), lambda i: (0, 0)),
        pl.BlockSpec((BN, 1), lambda i: (i, 0)),
    ],
    out_specs=pl.BlockSpec((G, D), lambda i: (0, 0)),
    out_shape=jax.ShapeDtypeStruct((G, D), jnp.float32),
)


def kernel(x, edge_index, batch, W_emb, b_emb, W1, b1, W2, b2):
    src = edge_index[0]
    dst = edge_index[1]
    srcoff = jnp.stack([src, src + NPAD]).reshape(2, NS, 2, NCH // 2, EC)
    dst3 = dst.reshape(NS, 2, NCH // 2, EC)
    dst4 = dst.reshape(2, NS, NCH2, EC)
    ones_rows = jnp.ones((EC, DW), jnp.float32)
    zeros_slab = jnp.zeros((NPAD // NS, DW), jnp.float32)

    deg_fn, spmm_fn = _sc_kernels()
    degp = deg_fn(dst4, ones_rows, zeros_slab)
    deg2d = degp[:N, 0:1] + degp[NPAD:NPAD + N, 0:1]

    t1 = _embed_fn(x, W_emb, b_emb.reshape(1, D), W1)
    hs1 = _scale_fn(t1, deg2d)
    agg1 = spmm_fn(hs1.reshape(2 * NPAD, H), srcoff, dst3)
    hs2 = _mid_fn(agg1.reshape(2, NPAD, H), deg2d, b1.reshape(1, D), W2)
    agg2 = spmm_fn(hs2.reshape(2 * NPAD, H), srcoff, dst3)
    return _pool_fn(agg2.reshape(2, NPAD, H), deg2d, b2.reshape(1, D),
                    batch.reshape(N, 1))


# speedup vs baseline: 1.0457x; 1.0457x over previous
import functools

import jax
import jax.numpy as jnp
from jax import lax
from jax.experimental import pallas as pl
from jax.experimental.pallas import tpu as pltpu
from jax.experimental.pallas import tpu_sc as plsc

N = 10000
E = 160000
D = 256
H = 128
G = 64
NPAD = 10240
NS = 16
EC = 125
NCH = E // NS // EC
NCH2 = E // 2 // NS // EC
DW = 32
BN = 2000
GRID = N // BN

@functools.cache
def _sc_kernels():
    mesh = plsc.VectorSubcoreMesh(core_axis_name="c", subcore_axis_name="s")
    deg_fn = functools.partial(
        pl.kernel,
        out_type=jax.ShapeDtypeStruct((2 * NPAD, DW), jnp.float32),
        mesh=mesh,
        scratch_types=[
            pltpu.VMEM_SHARED((NPAD, DW), jnp.float32),
            pltpu.VMEM((NCH2, EC), jnp.int32),
            pltpu.VMEM((EC, DW), jnp.float32),
        ],
    )(_deg_body)
    spmm_fn = functools.partial(
        pl.kernel,
        out_type=jax.ShapeDtypeStruct((2 * NPAD, H), jnp.float32),
        mesh=mesh,
        scratch_types=[
            pltpu.VMEM_SHARED((NPAD, H), jnp.float32),
            pltpu.VMEM((NCH // 2, EC), jnp.int32),
            pltpu.VMEM((NCH // 2, EC), jnp.int32),
            pltpu.VMEM((EC, H), jnp.float32),
            pltpu.VMEM((EC, H), jnp.float32),
            pltpu.SemaphoreType.DMA,
            pltpu.SemaphoreType.DMA,
        ],
    )(_spmm_body)
    return deg_fn, spmm_fn


def _deg_body(dst_hbm, ones_hbm, zeros_hbm, deg_hbm, degw, idxd, ones_v):
    c = lax.axis_index("c")
    s = lax.axis_index("s")
    r0 = s * (NPAD // NS)
    pltpu.sync_copy(zeros_hbm, degw.at[pl.ds(r0, NPAD // NS)])
    pltpu.sync_copy(ones_hbm, ones_v)
    pltpu.sync_copy(dst_hbm.at[c, s], idxd)
    plsc.subcore_barrier()

    def chunk(j, carry):
        pltpu.sync_copy(ones_v, degw.at[idxd.at[j]], add=True)
        return carry

    lax.fori_loop(0, NCH2, chunk, 0)
    plsc.subcore_barrier()
    pltpu.sync_copy(degw.at[pl.ds(r0, NPAD // NS)],
                    deg_hbm.at[pl.ds(c * NPAD + r0, NPAD // NS)])


def _spmm_body(hs_hbm, srcoff_hbm, dst_hbm, out_hbm, agg, idxs, idxd, rows0,
               rows1, sem0, sem1):
    c = lax.axis_index("c")
    s = lax.axis_index("s")
    r0 = s * (NPAD // NS)
    base = c * NPAD
    def seed(i, carry):
        pltpu.sync_copy(hs_hbm.at[pl.ds(base + r0 + i * 128, 128)],
                        agg.at[pl.ds(r0 + i * 128, 128)])
        return carry

    lax.fori_loop(0, (NPAD // NS) // 128, seed, 0)
    plsc.subcore_barrier()

    HCH = NCH // 2
    for h in range(2):
        pltpu.sync_copy(srcoff_hbm.at[c, s, h], idxs)
        pltpu.sync_copy(dst_hbm.at[s, h], idxd)
        pltpu.async_copy(hs_hbm.at[idxs.at[0]], rows0, sem0)
        pltpu.async_copy(hs_hbm.at[idxs.at[1]], rows1, sem1)

        def pair(k, carry):
            j0 = 2 * k
            j1 = j0 + 1
            j2 = jnp.minimum(j0 + 2, HCH - 1)
            j3 = jnp.minimum(j0 + 3, HCH - 1)
            pltpu.make_async_copy(hs_hbm.at[idxs.at[j0]], rows0, sem0).wait()
            pltpu.sync_copy(rows0, agg.at[idxd.at[j0]], add=True)
            pltpu.async_copy(hs_hbm.at[idxs.at[j2]], rows0, sem0)
            pltpu.make_async_copy(hs_hbm.at[idxs.at[j1]], rows1, sem1).wait()
            pltpu.sync_copy(rows1, agg.at[idxd.at[j1]], add=True)
            pltpu.async_copy(hs_hbm.at[idxs.at[j3]], rows1, sem1)
            return carry

        lax.fori_loop(0, HCH // 2, pair, 0)
        pltpu.make_async_copy(hs_hbm.at[idxs.at[HCH - 1]], rows0, sem0).wait()
        pltpu.make_async_copy(hs_hbm.at[idxs.at[HCH - 1]], rows1, sem1).wait()
    plsc.subcore_barrier()

    def wb(i, carry):
        pltpu.sync_copy(agg.at[pl.ds(r0 + i * 128, 128)],
                        out_hbm.at[pl.ds(base + r0 + i * 128, 128)])
        return carry

    lax.fori_loop(0, (NPAD // NS) // 128, wb, 0)


def _embed_body(x_ref, we_ref, be_ref, w1_ref, deg_ref, out_ref):
    dinv = lax.rsqrt(deg_ref[...] + 1.0)
    h0 = jnp.dot(x_ref[...], we_ref[...],
                 preferred_element_type=jnp.float32) + be_ref[...]
    t1 = jnp.dot(h0, w1_ref[...], preferred_element_type=jnp.float32)
    hs = t1 * dinv
    out_ref[0] = hs[:, :H]
    out_ref[1] = hs[:, H:]


_embed_fn = pl.pallas_call(
    _embed_body,
    grid=(GRID,),
    in_specs=[
        pl.BlockSpec((BN, D), lambda i: (i, 0)),
        pl.BlockSpec((D, D), lambda i: (0, 0)),
        pl.BlockSpec((1, D), lambda i: (0, 0)),
        pl.BlockSpec((D, D), lambda i: (0, 0)),
        pl.BlockSpec((BN, 1), lambda i: (i, 0)),
    ],
    out_specs=pl.BlockSpec((2, BN, H), lambda i: (0, i, 0)),
    out_shape=jax.ShapeDtypeStruct((2, NPAD, H), jnp.float32),
)


def _mid_body(agg_ref, deg_ref, b1_ref, w2_ref, out_ref):
    dinv = lax.rsqrt(deg_ref[...] + 1.0)
    a = agg_ref[...]
    h1 = jnp.concatenate([a[0], a[1]], axis=1)
    h1 = jax.nn.relu(h1 * dinv + b1_ref[...])
    t2 = jnp.dot(h1, w2_ref[...], preferred_element_type=jnp.float32)
    hs = t2 * dinv
    out_ref[0] = hs[:, :H]
    out_ref[1] = hs[:, H:]


_mid_fn = pl.pallas_call(
    _mid_body,
    grid=(GRID,),
    in_specs=[
        pl.BlockSpec((2, BN, H), lambda i: (0, i, 0)),
        pl.BlockSpec((BN, 1), lambda i: (i, 0)),
        pl.BlockSpec((1, D), lambda i: (0, 0)),
        pl.BlockSpec((D, D), lambda i: (0, 0)),
    ],
    out_specs=pl.BlockSpec((2, BN, H), lambda i: (0, i, 0)),
    out_shape=jax.ShapeDtypeStruct((2, NPAD, H), jnp.float32),
)


def _pool_body(agg_ref, deg_ref, b2_ref, batch_ref, out_ref):
    dinv = lax.rsqrt(deg_ref[...] + 1.0)
    a = agg_ref[...]
    h2 = jnp.concatenate([a[0], a[1]], axis=1)
    h2 = jax.nn.relu(h2 * dinv + b2_ref[...])
    gid = lax.broadcasted_iota(jnp.int32, (BN, G), 1)
    oh = (batch_ref[...] == gid).astype(jnp.float32)
    pooled = lax.dot_general(oh, h2, (((0,), (0,)), ((), ())),
                             preferred_element_type=jnp.float32)
    i = pl.program_id(0)

    @pl.when(i == 0)
    def _():
        out_ref[...] = pooled

    @pl.when(i > 0)
    def _():
        out_ref[...] += pooled


_pool_fn = pl.pallas_call(
    _pool_body,
    grid=(GRID,),
    in_specs=[
        pl.BlockSpec((2, BN, H), lambda i: (0, i, 0)),
        pl.BlockSpec((BN, 1), lambda i: (i, 0)),
        pl.BlockSpec((1, D), lambda i: (0, 0)),
        pl.BlockSpec((BN, 1), lambda i: (i, 0)),
    ],
    out_specs=pl.BlockSpec((G, D), lambda i: (0, 0)),
    out_shape=jax.ShapeDtypeStruct((G, D), jnp.float32),
)


def kernel(x, edge_index, batch, W_emb, b_emb, W1, b1, W2, b2):
    src = edge_index[0]
    dst = edge_index[1]
    srcoff = jnp.stack([src, src + NPAD]).reshape(2, NS, 2, NCH // 2, EC)
    dst3 = dst.reshape(NS, 2, NCH // 2, EC)
    dst4 = dst.reshape(2, NS, NCH2, EC)
    ones_rows = jnp.ones((EC, DW), jnp.float32)
    zeros_slab = jnp.zeros((NPAD // NS, DW), jnp.float32)

    deg_fn, spmm_fn = _sc_kernels()
    degp = deg_fn(dst4, ones_rows, zeros_slab)
    deg2d = degp[:N, 0:1] + degp[NPAD:NPAD + N, 0:1]

    hs1 = _embed_fn(x, W_emb, b_emb.reshape(1, D), W1, deg2d)
    agg1 = spmm_fn(hs1.reshape(2 * NPAD, H), srcoff, dst3)
    hs2 = _mid_fn(agg1.reshape(2, NPAD, H), deg2d, b1.reshape(1, D), W2)
    agg2 = spmm_fn(hs2.reshape(2 * NPAD, H), srcoff, dst3)
    return _pool_fn(agg2.reshape(2, NPAD, H), deg2d, b2.reshape(1, D),
                    batch.reshape(N, 1))
